# Initial kernel scaffold; baseline (speedup 1.0000x reference)
import jax
import jax.numpy as jnp
from jax.experimental import pallas as pl


def kernel(x, edge_index, batch, Wl0, Wr0, b0, Wl1, Wr1, b1, Wl2, Wr2, b2, Wout, bout):
    raise NotImplementedError("write your pallas kernel here")


# speedup vs baseline: 5.2794x; 5.2794x over previous
import functools

import jax
import jax.numpy as jnp
from jax import lax
from jax.experimental import pallas as pl
from jax.experimental.pallas import tpu as pltpu
from jax.experimental.pallas import tpu_sc as plsc

N = 10000
E = 320000
D = 128
G = 64
NCLS = 10

NC = 2
NS = 16
NW = NC * NS

CHUNK = 128
NCHUNKS = E // CHUNK

AGG_CHUNKS = NCHUNKS // NW
AGG_TAIL = NCHUNKS - AGG_CHUNKS * NW

DEG_CHUNKS = NCHUNKS // NS
DEG_TAIL = NCHUNKS - DEG_CHUNKS * NS

R_SUB = 624
R_LAST = N - R_SUB * (NS - 1)


def _slice_phase(s, copy_chunk):
    base = s * R_SUB

    @pl.when(s < NS - 1)
    def _():
        def blk(k, _):
            copy_chunk(base + k * 104, 104)
            return _
        lax.fori_loop(0, 6, blk, 0)

    @pl.when(s == NS - 1)
    def _():
        def blk(k, _):
            copy_chunk(base + k * 128, 128)
            return _
        lax.fori_loop(0, 5, blk, 0)


@functools.cache
def _make_agg():
    mesh = plsc.VectorSubcoreMesh(core_axis_name="c", subcore_axis_name="s")
    scratch = [
        pltpu.VMEM((CHUNK,), jnp.int32),
        pltpu.VMEM((1, CHUNK), jnp.int32),
        pltpu.VMEM((CHUNK, D), jnp.float32),
        pltpu.VMEM_SHARED((N, D), jnp.float32),
        pltpu.SemaphoreType.DMA,
    ]

    def body(h_hbm, src_hbm, dst_hbm, z128_hbm, acc_out, srcb, dstb, rows,
             acc_sh, sem):
        c = lax.axis_index("c")
        s = lax.axis_index("s")
        w = c * NS + s

        pltpu.sync_copy(z128_hbm, rows)
        _slice_phase(s, lambda r0, nr: pltpu.sync_copy(
            rows.at[pl.ds(0, nr)], acc_sh.at[pl.ds(r0, nr)]))
        plsc.subcore_barrier()

        def do_chunk(chunk_id):
            off = chunk_id * CHUNK
            pltpu.sync_copy(src_hbm.at[pl.ds(off, CHUNK)], srcb)
            pltpu.sync_copy(dst_hbm.at[pl.ds(off, CHUNK)], dstb.at[0])
            pltpu.async_copy(h_hbm.at[srcb], rows, sem).wait()
            pltpu.sync_copy(rows, acc_sh.at[dstb.at[0]], add=True)

        def loop_body(i, _):
            do_chunk(w * AGG_CHUNKS + i)
            return _

        lax.fori_loop(0, AGG_CHUNKS, loop_body, 0)

        @pl.when(w < AGG_TAIL)
        def _():
            do_chunk(NW * AGG_CHUNKS + w)

        plsc.subcore_barrier()

        def out_chunk(r0, nr):
            pltpu.sync_copy(acc_sh.at[pl.ds(r0, nr)], rows.at[pl.ds(0, nr)])
            pltpu.sync_copy(rows.at[pl.ds(0, nr)],
                            acc_out.at[c].at[pl.ds(r0, nr)])

        _slice_phase(s, out_chunk)

    return pl.kernel(body,
                     out_type=jax.ShapeDtypeStruct((NC, N, D), jnp.float32),
                     mesh=mesh, scratch_types=scratch, name="sage_agg")


@functools.cache
def _make_deg():
    mesh = plsc.VectorSubcoreMesh(core_axis_name="c", subcore_axis_name="s")
    scratch = [
        pltpu.VMEM((1, CHUNK), jnp.int32),
        pltpu.VMEM((CHUNK, D), jnp.float32),
        pltpu.VMEM((CHUNK, D), jnp.float32),
        pltpu.VMEM_SHARED((N, D), jnp.float32),
    ]

    def body(dst_hbm, z128_hbm, ones128_hbm, deg_out, dstb, onesr,
             zrows, deg_sh):
        c = lax.axis_index("c")
        s = lax.axis_index("s")
        w = c * NS + s

        pltpu.sync_copy(z128_hbm, zrows)
        _slice_phase(s, lambda r0, nr: pltpu.sync_copy(
            zrows.at[pl.ds(0, nr)], deg_sh.at[pl.ds(r0, nr)]))
        pltpu.sync_copy(ones128_hbm, onesr)
        plsc.subcore_barrier()

        def do_chunk(chunk_id):
            off = chunk_id * CHUNK
            pltpu.sync_copy(dst_hbm.at[pl.ds(off, CHUNK)], dstb.at[0])
            pltpu.sync_copy(onesr, deg_sh.at[dstb.at[0]], add=True)

        def loop_body(i, _):
            do_chunk(w * AGG_CHUNKS + i)
            return _

        lax.fori_loop(0, AGG_CHUNKS, loop_body, 0)

        @pl.when(w < AGG_TAIL)
        def _():
            do_chunk(NW * AGG_CHUNKS + w)

        plsc.subcore_barrier()

        def out_chunk(r0, nr):
            pltpu.sync_copy(deg_sh.at[pl.ds(r0, nr)], zrows.at[pl.ds(0, nr)])
            pltpu.sync_copy(zrows.at[pl.ds(0, nr)],
                            deg_out.at[c].at[pl.ds(r0, nr)])

        _slice_phase(s, out_chunk)

    return pl.kernel(body,
                     out_type=jax.ShapeDtypeStruct((NC, N, D), jnp.float32),
                     mesh=mesh, scratch_types=scratch, name="sage_deg")


BLK = 1000
NBLK = N // BLK


def _layer0_body(a0, a1, d0, d1, h, wl, wr, b, out, rout):
    deg = d0[:, 0:1] + d1[:, 0:1]
    r = 1.0 / jnp.maximum(deg, 1.0)
    mean = (a0[...] + a1[...]) * r
    y = (jnp.dot(mean, wl[...], preferred_element_type=jnp.float32)
         + jnp.dot(h[...], wr[...], preferred_element_type=jnp.float32)
         + b[...])
    out[...] = jnp.maximum(y, 0.0)
    rout[...] = r


def _layer_body(a0, a1, r, h, wl, wr, b, out):
    mean = (a0[...] + a1[...]) * r[...]
    y = (jnp.dot(mean, wl[...], preferred_element_type=jnp.float32)
         + jnp.dot(h[...], wr[...], preferred_element_type=jnp.float32)
         + b[...])
    out[...] = jnp.maximum(y, 0.0)


def _final_body(a0, a1, r, h, wl, wr, b, batch, wout, bout, logits,
                s_scr, c_scr):
    i = pl.program_id(0)
    mean = (a0[...] + a1[...]) * r[...]
    y = (jnp.dot(mean, wl[...], preferred_element_type=jnp.float32)
         + jnp.dot(h[...], wr[...], preferred_element_type=jnp.float32)
         + b[...])
    out = jnp.maximum(y, 0.0)
    onehot = (batch[...] == lax.broadcasted_iota(jnp.int32, (1, G), 1)
              ).astype(jnp.float32)
    contract = (((0,), (0,)), ((), ()))
    s_blk = lax.dot_general(onehot, out, contract,
                            preferred_element_type=jnp.float32)
    c_blk = lax.dot_general(onehot, jnp.ones((BLK, D), jnp.float32), contract,
                            preferred_element_type=jnp.float32)

    @pl.when(i == 0)
    def _():
        s_scr[...] = jnp.zeros((G, D), jnp.float32)
        c_scr[...] = jnp.zeros((G, D), jnp.float32)

    s_scr[...] += s_blk
    c_scr[...] += c_blk

    @pl.when(i == NBLK - 1)
    def _():
        pooled = s_scr[...] / jnp.maximum(c_scr[...], 1.0)
        logits[...] = (jnp.dot(pooled, wout[...],
                               preferred_element_type=jnp.float32) + bout[...])


def _row_spec(width):
    return pl.BlockSpec((BLK, width), lambda i: (i, 0))


def _const_spec(shape):
    return pl.BlockSpec(shape, lambda i: (0, 0))


_layer0_call = pl.pallas_call(
    _layer0_body,
    grid=(NBLK,),
    in_specs=[_row_spec(D), _row_spec(D), _row_spec(D), _row_spec(D),
              _row_spec(D), _const_spec((D, D)), _const_spec((D, D)),
              _const_spec((1, D))],
    out_specs=[_row_spec(D), _row_spec(1)],
    out_shape=[jax.ShapeDtypeStruct((N, D), jnp.float32),
               jax.ShapeDtypeStruct((N, 1), jnp.float32)],
)

_layer_call = pl.pallas_call(
    _layer_body,
    grid=(NBLK,),
    in_specs=[_row_spec(D), _row_spec(D), _row_spec(1), _row_spec(D),
              _const_spec((D, D)), _const_spec((D, D)), _const_spec((1, D))],
    out_specs=_row_spec(D),
    out_shape=jax.ShapeDtypeStruct((N, D), jnp.float32),
)

_final_call = pl.pallas_call(
    _final_body,
    grid=(NBLK,),
    in_specs=[_row_spec(D), _row_spec(D), _row_spec(1), _row_spec(D),
              _const_spec((D, D)), _const_spec((D, D)), _const_spec((1, D)),
              _row_spec(1), _const_spec((D, NCLS)), _const_spec((1, NCLS))],
    out_specs=_const_spec((G, NCLS)),
    out_shape=jax.ShapeDtypeStruct((G, NCLS), jnp.float32),
    scratch_shapes=[pltpu.VMEM((G, D), jnp.float32),
                    pltpu.VMEM((G, D), jnp.float32)],
)


def kernel(x, edge_index, batch, Wl0, Wr0, b0, Wl1, Wr1, b1, Wl2, Wr2, b2,
           Wout, bout):
    src = edge_index[0]
    dst = edge_index[1]
    z128 = jnp.zeros((CHUNK, D), jnp.float32)
    ones128 = jnp.ones((CHUNK, D), jnp.float32)
    b0r = jnp.reshape(b0, (1, D))
    b1r = jnp.reshape(b1, (1, D))
    b2r = jnp.reshape(b2, (1, D))
    boutr = jnp.reshape(bout, (1, NCLS))
    batch2 = jnp.reshape(batch, (N, 1))

    deg = _make_deg()(dst, z128, ones128)
    agg = _make_agg()

    acc = agg(x, src, dst, z128)
    h1, r = _layer0_call(acc[0], acc[1], deg[0], deg[1], x, Wl0, Wr0, b0r)
    acc1 = agg(h1, src, dst, z128)
    h2 = _layer_call(acc1[0], acc1[1], r, h1, Wl1, Wr1, b1r)
    acc2 = agg(h2, src, dst, z128)
    logits = _final_call(acc2[0], acc2[1], r, h2, Wl2, Wr2, b2r,
                         batch2, Wout, boutr)
    return logits
